# baseline (speedup 1.0000x reference)
import functools

import jax
import jax.numpy as jnp
from jax import lax
from jax.experimental import pallas as pl
from jax.experimental.pallas import tpu as pltpu
from jax.experimental.pallas import tpu_sc as plsc

G = 1024
U = 20
L = 50
D = 64
FACTOR = 0.5

NC = 2
NS = 16
NW = NC * NS
GPW = G // NW

CHUNK = 100
NCHUNK = (U * L) // CHUNK
TCH = GPW * NCHUNK
NBUF = 5
NK = D // 16


def _body(gbi_hbm, cnt_hbm, gu_hbm, tgt_hbm, sim_hbm, utab_hbm, itab_hbm,
          out_hbm,
          idx_v, cnt_v, rows_v, gu_v, tgt_v, trows_v, urows_v, srows_v,
          out_v, sem0, sem1, sem2, sem3, sem4, semg, sems_t):
    sems = [sem0, sem1, sem2, sem3, sem4]
    wid = lax.axis_index("s") * NC + lax.axis_index("c")
    gbase = wid * GPW

    pltpu.sync_copy(gu_hbm.at[pl.ds(gbase, GPW)], gu_v)
    pltpu.sync_copy(tgt_hbm.at[pl.ds(gbase, GPW)], tgt_v)
    cp_t = pltpu.async_copy(sim_hbm.at[tgt_v], trows_v, semg)
    pltpu.sync_copy(gbi_hbm.at[pl.ds(gbase, GPW)], idx_v)
    pltpu.sync_copy(cnt_hbm.at[pl.ds(gbase, GPW)], cnt_v)
    cp_t.wait()

    def urows_start(gl, pb):
        pltpu.async_copy(utab_hbm.at[gu_v.at[gl]], urows_v.at[pb], sems_t)
        pltpu.async_copy(sim_hbm.at[gu_v.at[gl]], srows_v.at[pb], sems_t)

    def urows_wait(gl, pb):
        pltpu.make_async_copy(utab_hbm.at[gu_v.at[gl]], urows_v.at[pb],
                              sems_t).wait()
        pltpu.make_async_copy(sim_hbm.at[gu_v.at[gl]], srows_v.at[pb],
                              sems_t).wait()

    def chunk_start(gl, j, bi):
        return pltpu.async_copy(itab_hbm.at[idx_v.at[gl, j]],
                                rows_v.at[bi], sems[bi])

    urows_start(0, 0)
    for bi in range(NBUF):
        chunk_start(0, bi, bi)

    zeros4 = tuple(jnp.zeros((16,), jnp.float32) for _ in range(NK))

    @pl.loop(0, TCH // NBUF, init_carry=zeros4)
    def _super(si, og):
        og = list(og)
        for bi in range(NBUF):
            c = si * NBUF + bi
            gl = lax.div(c, NCHUNK)
            j = lax.rem(c, NCHUNK)
            pb = lax.rem(gl, 2)

            @pl.when(j == 0)
            def _():
                urows_wait(gl, pb)

                @pl.when(gl + 1 < GPW)
                def _():
                    urows_start(gl + 1, 1 - pb)

            pltpu.make_async_copy(itab_hbm.at[idx_v.at[gl, j]],
                                  rows_v.at[bi], sems[bi]).wait()

            ts = [trows_v[gl, pl.ds(16 * k, 16)] for k in range(NK)]
            for u2 in range(2):
                u = 2 * j + u2
                cbase = u * L
                rbase = u2 * L

                def _tblock(t, acc):
                    cv = cnt_v[gl, pl.ds(cbase + t * 16, 16)]
                    acc = list(acc)
                    for i in range(16):
                        cc = cv[i]
                        for k in range(NK):
                            acc[k] = acc[k] + rows_v[
                                bi, rbase + t * 16 + i,
                                pl.ds(16 * k, 16)] * cc
                    return tuple(acc)

                acc = list(lax.fori_loop(0, 3, _tblock, zeros4))
                cvt = cnt_v[gl, pl.ds(cbase + 34, 16)]
                for i in range(2):
                    cc = cvt[14 + i]
                    for k in range(NK):
                        acc[k] = acc[k] + rows_v[
                            bi, rbase + 48 + i, pl.ds(16 * k, 16)] * cc

                s = jnp.float32(0.0)
                for k in range(NK):
                    s = s + jnp.sum(ts[k] * srows_v[pb, u, pl.ds(16 * k, 16)])
                s = s * FACTOR
                for k in range(NK):
                    og[k] = og[k] + acc[k] * urows_v[
                        pb, u, pl.ds(16 * k, 16)] * s

            @pl.when(j == NCHUNK - 1)
            def _():
                for k in range(NK):
                    out_v[gl, pl.ds(16 * k, 16)] = og[k]

            og = [jnp.where(j == NCHUNK - 1, jnp.zeros((16,), jnp.float32),
                            og[k]) for k in range(NK)]

            @pl.when(c + NBUF < TCH)
            def _():
                cn = c + NBUF
                chunk_start(lax.div(cn, NCHUNK), lax.rem(cn, NCHUNK), bi)
        return tuple(og)

    pltpu.sync_copy(out_v, out_hbm.at[pl.ds(gbase, GPW)])


@functools.cache
def _make_sc_call():
  return pl.kernel(
    _body,
    out_type=jax.ShapeDtypeStruct((G, D), jnp.float32),
    mesh=plsc.VectorSubcoreMesh(core_axis_name="c", subcore_axis_name="s",
                                num_cores=NC, num_subcores=NS),
    compiler_params=pltpu.CompilerParams(needs_layout_passes=False,
                                         use_tc_tiling_on_sc=False),
    scratch_types=[
        pltpu.VMEM((GPW, NCHUNK, CHUNK), jnp.int32),
        pltpu.VMEM((GPW, U * L), jnp.float32),
        pltpu.VMEM((NBUF, CHUNK, D), jnp.float32),
        pltpu.VMEM((GPW, U), jnp.int32),
        pltpu.VMEM((GPW,), jnp.int32),
        pltpu.VMEM((GPW, D), jnp.float32),
        pltpu.VMEM((2, U, D), jnp.float32),
        pltpu.VMEM((2, U, D), jnp.float32),
        pltpu.VMEM((GPW, D), jnp.float32),
        pltpu.SemaphoreType.DMA,
        pltpu.SemaphoreType.DMA,
        pltpu.SemaphoreType.DMA,
        pltpu.SemaphoreType.DMA,
        pltpu.SemaphoreType.DMA,
        pltpu.SemaphoreType.DMA,
        pltpu.SemaphoreType.DMA,
    ],
  )


@jax.jit
def kernel(group_user, group_behavior_ids, group_behavior_counts,
           target_user, similarity_vec, user_table, item_table):
    gbi = group_behavior_ids.astype(jnp.int32).reshape(G, NCHUNK, CHUNK)
    cnt = group_behavior_counts.reshape(G, U * L)
    gu = group_user.astype(jnp.int32)
    tgt = target_user.astype(jnp.int32)
    return _make_sc_call()(gbi, cnt, gu, tgt, similarity_vec, user_table,
                           item_table)
